# Initial kernel scaffold; baseline (speedup 1.0000x reference)
import jax
import jax.numpy as jnp
from jax.experimental import pallas as pl


def kernel(x, params):
    raise NotImplementedError("write your pallas kernel here")


# speedup vs baseline: 1.1778x; 1.1778x over previous
import jax
import jax.numpy as jnp
from jax.experimental import pallas as pl
from jax.experimental.pallas import tpu as pltpu

_ACTS = ['gelu', 'silu', 'relu', 'leaky_relu']
_DEPTHS = [1, 2, 3]
_HIDS = [1024, 2048, 3072]

_BT = 512
_BK = 1024


def _cfg(i):
    return _ACTS[i % 4], _DEPTHS[i % 3], _HIDS[i % 3]


def _apply_act(name, h):
    if name == 'gelu':
        return 0.5 * h * (1.0 + jax.lax.erf(h * (2.0 ** -0.5)))
    if name == 'silu':
        return h * (1.0 / (1.0 + jnp.exp(-h)))
    if name == 'relu':
        return jnp.maximum(h, 0.0)
    return jnp.where(h >= 0, h, 0.01 * h)


def _layernorm(h, g, b):
    m = jnp.mean(h, axis=-1, keepdims=True)
    v = jnp.mean((h - m) ** 2, axis=-1, keepdims=True)
    return (h - m) / jnp.sqrt(v + 1e-5) * g + b


def _gate_body(x_ref, gw_ref, gb_ref, w_ref):
    logits = jnp.dot(x_ref[...], gw_ref[...],
                     preferred_element_type=jnp.float32) + gb_ref[...]
    m = jnp.max(logits, axis=-1, keepdims=True)
    ex = jnp.exp(logits - m)
    s = ex / jnp.sum(ex, axis=-1, keepdims=True)
    n, e = s.shape
    col = jax.lax.broadcasted_iota(jnp.int32, (n, e), 1)
    v1 = jnp.max(s, axis=-1, keepdims=True)
    i1 = jnp.min(jnp.where(s == v1, col, e), axis=-1, keepdims=True)
    oh1 = col == i1
    s2 = jnp.where(oh1, -1.0, s)
    v2 = jnp.max(s2, axis=-1, keepdims=True)
    i2 = jnp.min(jnp.where(s2 == v2, col, e), axis=-1, keepdims=True)
    oh2 = col == i2
    w_ref[...] = jnp.where(oh1, v1, 0.0) + jnp.where(oh2, v2, 0.0)


def _gate(x, gw, gb):
    n = x.shape[0]
    ne = gw.shape[1]
    return pl.pallas_call(
        _gate_body,
        out_shape=jax.ShapeDtypeStruct((n, ne), jnp.float32),
    )(x, gw, gb.reshape(1, ne))


def _hidden_layer(h, W, b, g, beta, act):
    T, K = h.shape
    N = W.shape[1]
    bt, bk = min(_BT, T), min(_BK, K)
    nk = K // bk

    def body(x_ref, w_ref, b_ref, g_ref, bt_ref, o_ref, acc):
        k = pl.program_id(1)
        prod = jnp.dot(x_ref[...], w_ref[...].astype(jnp.bfloat16),
                       preferred_element_type=jnp.float32)

        @pl.when(k == 0)
        def _():
            acc[...] = prod

        @pl.when(k > 0)
        def _():
            acc[...] += prod

        @pl.when(k == nk - 1)
        def _():
            hh = acc[...] + b_ref[...]
            hh = _apply_act(act, hh)
            hh = _layernorm(hh, g_ref[...], bt_ref[...])
            o_ref[...] = hh.astype(jnp.bfloat16)

    return pl.pallas_call(
        body,
        grid=(T // bt, nk),
        in_specs=[
            pl.BlockSpec((bt, bk), lambda t, k: (t, k)),
            pl.BlockSpec((bk, N), lambda t, k: (k, 0)),
            pl.BlockSpec((1, N), lambda t, k: (0, 0)),
            pl.BlockSpec((1, N), lambda t, k: (0, 0)),
            pl.BlockSpec((1, N), lambda t, k: (0, 0)),
        ],
        out_specs=pl.BlockSpec((bt, N), lambda t, k: (t, 0)),
        out_shape=jax.ShapeDtypeStruct((T, N), jnp.bfloat16),
        scratch_shapes=[pltpu.VMEM((bt, N), jnp.float32)],
    )(h, W, b.reshape(1, N), g.reshape(1, N), beta.reshape(1, N))


def _final_layer(prev, h, W, b, g, beta, wcol):
    T, K = h.shape
    N = W.shape[1]
    bt, bk = min(_BT, T), min(_BK, K)
    nk = K // bk

    def body(p_ref, x_ref, w_ref, b_ref, g_ref, bt_ref, wc_ref, o_ref, acc):
        k = pl.program_id(1)
        prod = jnp.dot(x_ref[...], w_ref[...].astype(jnp.bfloat16),
                       preferred_element_type=jnp.float32)

        @pl.when(k == 0)
        def _():
            acc[...] = prod

        @pl.when(k > 0)
        def _():
            acc[...] += prod

        @pl.when(k == nk - 1)
        def _():
            hh = acc[...] + b_ref[...]
            hh = _layernorm(hh, g_ref[...], bt_ref[...])
            o_ref[...] = p_ref[...] + wc_ref[...] * hh

    return pl.pallas_call(
        body,
        grid=(T // bt, nk),
        in_specs=[
            pl.BlockSpec((bt, N), lambda t, k: (t, 0)),
            pl.BlockSpec((bt, bk), lambda t, k: (t, k)),
            pl.BlockSpec((bk, N), lambda t, k: (k, 0)),
            pl.BlockSpec((1, N), lambda t, k: (0, 0)),
            pl.BlockSpec((1, N), lambda t, k: (0, 0)),
            pl.BlockSpec((1, N), lambda t, k: (0, 0)),
            pl.BlockSpec((bt, 1), lambda t, k: (t, 0)),
        ],
        out_specs=pl.BlockSpec((bt, N), lambda t, k: (t, 0)),
        out_shape=jax.ShapeDtypeStruct((T, N), jnp.float32),
        scratch_shapes=[pltpu.VMEM((bt, N), jnp.float32)],
        input_output_aliases={0: 0},
    )(prev, h, W, b.reshape(1, N), g.reshape(1, N), beta.reshape(1, N), wcol)


def kernel(x, params):
    experts = params['experts']
    n_exp = len(experts)
    weights = _gate(x, params['gate_W'], params['gate_b'])
    xb = x.astype(jnp.bfloat16)
    out = jnp.zeros_like(x)
    for i in range(n_exp):
        act, depth, _hid = _cfg(i)
        h = xb
        for j in range(depth):
            p = experts[i][j]
            h = _hidden_layer(h, p['W'], p['b'], p['g'], p['beta'], act)
        p = experts[i][depth]
        out = _final_layer(out, h, p['W'], p['b'], p['g'], p['beta'],
                           weights[:, i:i + 1])
    return out


# speedup vs baseline: 1.8430x; 1.5648x over previous
import functools

import jax
import jax.numpy as jnp
from jax import lax
from jax.experimental import pallas as pl
from jax.experimental.pallas import tpu as pltpu
from jax.experimental.pallas import tpu_sc as plsc

_ACTS = ['gelu', 'silu', 'relu', 'leaky_relu']
_DEPTHS = [1, 2, 3]
_HIDS = [1024, 2048, 3072]

_B = 256
_NEB = 8
_ECAP = (_NEB + 1) * _B
_NE = 8
_SLOTS = _NE * _ECAP


def _cfg(i):
    return _ACTS[i % 4], _DEPTHS[i % 3], _HIDS[i % 3]


def _apply_act(name, h):
    if name == 'gelu':
        return 0.5 * h * (1.0 + jax.lax.erf(h * (2.0 ** -0.5)))
    if name == 'silu':
        return h * (1.0 / (1.0 + jnp.exp(-h)))
    if name == 'relu':
        return jnp.maximum(h, 0.0)
    return jnp.where(h >= 0, h, 0.01 * h)


def _layernorm(h, g, b):
    m = jnp.mean(h, axis=-1, keepdims=True)
    v = jnp.mean((h - m) ** 2, axis=-1, keepdims=True)
    return (h - m) / jnp.sqrt(v + 1e-5) * g + b


def _gate_body(x_ref, gw_ref, gb_ref, i_ref, v_ref):
    logits = jnp.dot(x_ref[...], gw_ref[...],
                     preferred_element_type=jnp.float32) + gb_ref[...]
    m = jnp.max(logits, axis=-1, keepdims=True)
    ex = jnp.exp(logits - m)
    s = ex / jnp.sum(ex, axis=-1, keepdims=True)
    n, e = s.shape
    col = jax.lax.broadcasted_iota(jnp.int32, (n, e), 1)
    v1 = jnp.max(s, axis=-1, keepdims=True)
    i1 = jnp.min(jnp.where(s == v1, col, e), axis=-1, keepdims=True)
    s2 = jnp.where(col == i1, -1.0, s)
    v2 = jnp.max(s2, axis=-1, keepdims=True)
    i2 = jnp.min(jnp.where(s2 == v2, col, e), axis=-1, keepdims=True)
    i_ref[...] = jnp.concatenate([i1, i2], axis=1)
    v_ref[...] = jnp.concatenate([v1, v2], axis=1)


def _gate(x, gw, gb):
    n = x.shape[0]
    ne = gw.shape[1]
    return pl.pallas_call(
        _gate_body,
        out_shape=[jax.ShapeDtypeStruct((n, 2), jnp.int32),
                   jax.ShapeDtypeStruct((n, 2), jnp.float32)],
    )(x, gw, gb.reshape(1, ne))


def _meta_body(nch, e_ref, pos_ref, nb_ref):
    ch = 512
    iota8 = lax.broadcasted_iota(jnp.int32, (1, _NE), 1)

    def cpass(c, carry):
        ev = e_ref[pl.ds(c * ch, ch), :]
        oh = (ev == iota8).astype(jnp.float32)
        return carry + jnp.sum(oh, axis=0, keepdims=True)

    counts = lax.fori_loop(0, nch, cpass, jnp.zeros((1, _NE), jnp.float32))
    nb_ref[...] = jnp.floor((counts + (_B - 1)) * (1.0 / _B)).astype(jnp.int32)

    rr = lax.broadcasted_iota(jnp.int32, (ch, ch), 0)
    cc = lax.broadcasted_iota(jnp.int32, (ch, ch), 1)
    tril = (cc < rr).astype(jnp.float32)
    base8 = (iota8 * _ECAP).astype(jnp.float32)

    def rpass(c, carry):
        ev = e_ref[pl.ds(c * ch, ch), :]
        oh = (ev == iota8).astype(jnp.float32)
        ranks = jnp.dot(tril, oh, preferred_element_type=jnp.float32) + carry
        posv = jnp.sum(oh * (ranks + base8), axis=1, keepdims=True)
        pos_ref[pl.ds(c * ch, ch), :] = posv.astype(jnp.int32)
        return carry + jnp.sum(oh, axis=0, keepdims=True)

    lax.fori_loop(0, nch, rpass, jnp.zeros((1, _NE), jnp.float32))


def _meta(e2):
    a = e2.shape[0]
    return pl.pallas_call(
        functools.partial(_meta_body, a // 512),
        out_shape=[jax.ShapeDtypeStruct((a, 1), jnp.int32),
                   jax.ShapeDtypeStruct((1, _NE), jnp.int32)],
    )(e2)


def _sc_dispatch(x, p0, p1):
    n, d = x.shape
    cpt = n // 32
    mesh = plsc.VectorSubcoreMesh(core_axis_name="c", subcore_axis_name="s")

    @functools.partial(
        pl.kernel, mesh=mesh,
        out_type=jax.ShapeDtypeStruct((_SLOTS, d), jnp.float32),
        scratch_types=[
            pltpu.VMEM((cpt,), jnp.int32),
            pltpu.VMEM((cpt,), jnp.int32),
            pltpu.VMEM((cpt, d), jnp.float32),
            pltpu.SemaphoreType.DMA,
        ],
    )
    def k(x_hbm, p0_hbm, p1_hbm, xs_hbm, i0_v, i1_v, rows_v, sem):
        wid = lax.axis_index("s") * 2 + lax.axis_index("c")
        base = wid * cpt
        pltpu.sync_copy(p0_hbm.at[pl.ds(base, cpt)], i0_v)
        pltpu.sync_copy(p1_hbm.at[pl.ds(base, cpt)], i1_v)
        pltpu.sync_copy(x_hbm.at[pl.ds(base, cpt)], rows_v)
        pltpu.async_copy(rows_v, xs_hbm.at[i0_v], sem).wait()
        pltpu.async_copy(rows_v, xs_hbm.at[i1_v], sem).wait()

    return k(x, p0, p1)


def _sc_combine(ys, p0, p1):
    n = p0.shape[0]
    d = ys.shape[1]
    cpt = n // 32
    half = cpt // 2
    mesh = plsc.VectorSubcoreMesh(core_axis_name="c", subcore_axis_name="s")

    @functools.partial(
        pl.kernel, mesh=mesh,
        out_type=(jax.ShapeDtypeStruct((n, d), jnp.float32),
                  jax.ShapeDtypeStruct((n, d), jnp.float32)),
        scratch_types=[
            pltpu.VMEM((half,), jnp.int32),
            pltpu.VMEM((half, d), jnp.float32),
            pltpu.SemaphoreType.DMA,
        ],
    )
    def k(ys_hbm, p0_hbm, p1_hbm, g0_hbm, g1_hbm, i_v, buf_v, sem):
        wid = lax.axis_index("s") * 2 + lax.axis_index("c")
        base = wid * cpt
        for c in range(2):
            b2 = base + c * half
            pltpu.sync_copy(p0_hbm.at[pl.ds(b2, half)], i_v)
            pltpu.async_copy(ys_hbm.at[i_v], buf_v, sem).wait()
            pltpu.sync_copy(buf_v, g0_hbm.at[pl.ds(b2, half)])
            pltpu.sync_copy(p1_hbm.at[pl.ds(b2, half)], i_v)
            pltpu.async_copy(ys_hbm.at[i_v], buf_v, sem).wait()
            pltpu.sync_copy(buf_v, g1_hbm.at[pl.ds(b2, half)])

    return k(ys, p0, p1)


def _ragged_hidden(h_in, W, b, g, beta, act, e, nbf, first):
    K, N = W.shape
    nk = K // 1024
    nbe = jnp.maximum(nbf[e], 1)

    def xmap(j, k, nb):
        jj = jnp.where(j < nb[e], j, _NEB)
        return (_NE * 0 + 9 * e + jj, k) if first else (jj, k)

    def omap(j, k, nb):
        return (jnp.where(j < nb[e], j, _NEB), 0)

    def body(nb_ref, x_ref, w_ref, b_ref, g_ref, bt_ref, o_ref, acc):
        k = pl.program_id(1)
        xv = x_ref[...]
        if first:
            xv = xv.astype(jnp.bfloat16)
        prod = jnp.dot(xv, w_ref[...].astype(jnp.bfloat16),
                       preferred_element_type=jnp.float32)

        @pl.when(k == 0)
        def _():
            acc[...] = prod

        @pl.when(k > 0)
        def _():
            acc[...] += prod

        @pl.when(k == nk - 1)
        def _():
            hh = acc[...] + b_ref[...]
            hh = _apply_act(act, hh)
            hh = _layernorm(hh, g_ref[...], bt_ref[...])
            o_ref[...] = hh.astype(jnp.bfloat16)

    grid_spec = pltpu.PrefetchScalarGridSpec(
        num_scalar_prefetch=1,
        grid=(nbe, nk),
        in_specs=[
            pl.BlockSpec((_B, 1024), xmap),
            pl.BlockSpec((1024, N), lambda j, k, nb: (k, 0)),
            pl.BlockSpec((1, N), lambda j, k, nb: (0, 0)),
            pl.BlockSpec((1, N), lambda j, k, nb: (0, 0)),
            pl.BlockSpec((1, N), lambda j, k, nb: (0, 0)),
        ],
        out_specs=pl.BlockSpec((_B, N), omap),
        scratch_shapes=[pltpu.VMEM((_B, N), jnp.float32)],
    )
    return pl.pallas_call(
        body,
        grid_spec=grid_spec,
        out_shape=jax.ShapeDtypeStruct((_ECAP, N), jnp.bfloat16),
    )(nbf, h_in, W, b.reshape(1, N), g.reshape(1, N), beta.reshape(1, N))


def _ragged_final(ys_prev, h_in, W, b, g, beta, e, nbf):
    K, N = W.shape
    nk = K // 1024
    nbe = jnp.maximum(nbf[e], 1)

    def hmap(j, k, nb):
        return (jnp.where(j < nb[e], j, _NEB), k)

    def omap(j, k, nb):
        return (9 * e + jnp.where(j < nb[e], j, _NEB), 0)

    def body(nb_ref, *refs):
        if ys_prev is None:
            h_ref, w_ref, b_ref, g_ref, bt_ref, o_ref, acc = refs
        else:
            _yp, h_ref, w_ref, b_ref, g_ref, bt_ref, o_ref, acc = refs
        k = pl.program_id(1)
        prod = jnp.dot(h_ref[...], w_ref[...].astype(jnp.bfloat16),
                       preferred_element_type=jnp.float32)

        @pl.when(k == 0)
        def _():
            acc[...] = prod

        @pl.when(k > 0)
        def _():
            acc[...] += prod

        @pl.when(k == nk - 1)
        def _():
            hh = acc[...] + b_ref[...]
            o_ref[...] = _layernorm(hh, g_ref[...], bt_ref[...])

    in_specs = [
        pl.BlockSpec((_B, 1024), hmap),
        pl.BlockSpec((1024, N), lambda j, k, nb: (k, 0)),
        pl.BlockSpec((1, N), lambda j, k, nb: (0, 0)),
        pl.BlockSpec((1, N), lambda j, k, nb: (0, 0)),
        pl.BlockSpec((1, N), lambda j, k, nb: (0, 0)),
    ]
    args = [nbf, h_in, W, b.reshape(1, N), g.reshape(1, N), beta.reshape(1, N)]
    aliases = {}
    if ys_prev is not None:
        in_specs.insert(0, pl.BlockSpec(memory_space=pl.ANY))
        args.insert(1, ys_prev)
        aliases = {1: 0}
    grid_spec = pltpu.PrefetchScalarGridSpec(
        num_scalar_prefetch=1,
        grid=(nbe, nk),
        in_specs=in_specs,
        out_specs=pl.BlockSpec((_B, N), omap),
        scratch_shapes=[pltpu.VMEM((_B, N), jnp.float32)],
    )
    return pl.pallas_call(
        body,
        grid_spec=grid_spec,
        out_shape=jax.ShapeDtypeStruct((_SLOTS, N), jnp.float32),
        input_output_aliases=aliases,
    )(*args)


def _mix_body(v_ref, a_ref, b_ref, o_ref):
    v = v_ref[...]
    o_ref[...] = v[:, 0:1] * a_ref[...] + v[:, 1:2] * b_ref[...]


def _mix(val2, g0, g1):
    n, d = g0.shape
    bt = 512
    return pl.pallas_call(
        _mix_body,
        grid=(n // bt,),
        in_specs=[
            pl.BlockSpec((bt, 2), lambda t: (t, 0)),
            pl.BlockSpec((bt, d), lambda t: (t, 0)),
            pl.BlockSpec((bt, d), lambda t: (t, 0)),
        ],
        out_specs=pl.BlockSpec((bt, d), lambda t: (t, 0)),
        out_shape=jax.ShapeDtypeStruct((n, d), jnp.float32),
    )(val2, g0, g1)


def kernel(x, params):
    n = x.shape[0]
    idx2, val2 = _gate(x, params['gate_W'], params['gate_b'])
    pos, nb8 = _meta(idx2.reshape(2 * n, 1))
    pos2 = pos.reshape(n, 2)
    p0 = pos2[:, 0]
    p1 = pos2[:, 1]
    nbf = nb8.reshape(_NE)
    xs = _sc_dispatch(x, p0, p1)
    ys = None
    for e in range(_NE):
        act, depth, _hid = _cfg(e)
        layers = params['experts'][e]
        h = _ragged_hidden(xs, layers[0]['W'], layers[0]['b'], layers[0]['g'],
                           layers[0]['beta'], act, e, nbf, first=True)
        for j in range(1, depth):
            h = _ragged_hidden(h, layers[j]['W'], layers[j]['b'],
                               layers[j]['g'], layers[j]['beta'],
                               act, e, nbf, first=False)
        p = layers[depth]
        ys = _ragged_final(ys, h, p['W'], p['b'], p['g'], p['beta'], e, nbf)
    g0, g1 = _sc_combine(ys, p0, p1)
    return _mix(val2, g0, g1)


# speedup vs baseline: 2.1237x; 1.1523x over previous
import functools

import jax
import jax.numpy as jnp
from jax import lax
from jax.experimental import pallas as pl
from jax.experimental.pallas import tpu as pltpu
from jax.experimental.pallas import tpu_sc as plsc

_ACTS = ['gelu', 'silu', 'relu', 'leaky_relu']
_DEPTHS = [1, 2, 3]
_HIDS = [1024, 2048, 3072]

_B = 256
_NEB = 8
_ECAP = (_NEB + 1) * _B
_NE = 8
_SLOTS = _NE * _ECAP


def _cfg(i):
    return _ACTS[i % 4], _DEPTHS[i % 3], _HIDS[i % 3]


def _apply_act(name, h):
    if name == 'gelu':
        return 0.5 * h * (1.0 + jax.lax.erf(h * (2.0 ** -0.5)))
    if name == 'silu':
        return h * (1.0 / (1.0 + jnp.exp(-h)))
    if name == 'relu':
        return jnp.maximum(h, 0.0)
    return jnp.where(h >= 0, h, 0.01 * h)


def _layernorm(h, g, b):
    m = jnp.mean(h, axis=-1, keepdims=True)
    v = jnp.mean((h - m) ** 2, axis=-1, keepdims=True)
    return (h - m) / jnp.sqrt(v + 1e-5) * g + b


def _gate_meta_body(x_ref, gw_ref, gb_ref, v_ref, pos_ref, nb_ref):
    logits = jnp.dot(x_ref[...], gw_ref[...],
                     preferred_element_type=jnp.float32) + gb_ref[...]
    m = jnp.max(logits, axis=-1, keepdims=True)
    ex = jnp.exp(logits - m)
    s = ex / jnp.sum(ex, axis=-1, keepdims=True)
    n, e = s.shape
    col = jax.lax.broadcasted_iota(jnp.int32, (n, e), 1)
    v1 = jnp.max(s, axis=-1, keepdims=True)
    i1 = jnp.min(jnp.where(s == v1, col, e), axis=-1, keepdims=True)
    s2 = jnp.where(col == i1, -1.0, s)
    v2 = jnp.max(s2, axis=-1, keepdims=True)
    i2 = jnp.min(jnp.where(s2 == v2, col, e), axis=-1, keepdims=True)
    v_ref[...] = jnp.concatenate([v1, v2], axis=1)

    ch = 1024
    iota8 = jax.lax.broadcasted_iota(jnp.int32, (1, _NE), 1)
    rr = jax.lax.broadcasted_iota(jnp.int32, (ch, ch), 0)
    cc = jax.lax.broadcasted_iota(jnp.int32, (ch, ch), 1)
    tril = (cc < rr).astype(jnp.float32)
    base8 = (iota8 * _ECAP).astype(jnp.float32)
    carry = jnp.zeros((1, _NE), jnp.float32)
    nch = (2 * n) // ch
    for c in range(nch):
        src = i1 if c < nch // 2 else i2
        lo = (c % (nch // 2)) * ch
        ev = src[lo:lo + ch, :]
        oh = (ev == iota8).astype(jnp.float32)
        ranks = jnp.dot(tril, oh, preferred_element_type=jnp.float32) + carry
        posv = jnp.sum(oh * (ranks + base8), axis=1, keepdims=True)
        pos_ref[pl.ds(c * ch, ch), :] = posv.astype(jnp.int32)
        carry = carry + jnp.sum(oh, axis=0, keepdims=True)
    nb_ref[...] = jnp.floor((carry + (_B - 1)) * (1.0 / _B)).astype(jnp.int32)


def _gate_meta(x, gw, gb):
    n = x.shape[0]
    ne = gw.shape[1]
    return pl.pallas_call(
        _gate_meta_body,
        out_shape=[jax.ShapeDtypeStruct((n, 2), jnp.float32),
                   jax.ShapeDtypeStruct((2 * n, 1), jnp.int32),
                   jax.ShapeDtypeStruct((1, _NE), jnp.int32)],
    )(x, gw, gb.reshape(1, ne))


def _sc_dispatch(x, p0, p1):
    n, d = x.shape
    cpt = n // 32
    mesh = plsc.VectorSubcoreMesh(core_axis_name="c", subcore_axis_name="s")

    @functools.partial(
        pl.kernel, mesh=mesh,
        out_type=jax.ShapeDtypeStruct((_SLOTS, d), jnp.float32),
        scratch_types=[
            pltpu.VMEM((cpt,), jnp.int32),
            pltpu.VMEM((cpt,), jnp.int32),
            pltpu.VMEM((cpt, d), jnp.float32),
            pltpu.SemaphoreType.DMA,
        ],
    )
    def k(x_hbm, p0_hbm, p1_hbm, xs_hbm, i0_v, i1_v, rows_v, sem):
        wid = lax.axis_index("s") * 2 + lax.axis_index("c")
        base = wid * cpt
        pltpu.sync_copy(p0_hbm.at[pl.ds(base, cpt)], i0_v)
        pltpu.sync_copy(p1_hbm.at[pl.ds(base, cpt)], i1_v)
        pltpu.sync_copy(x_hbm.at[pl.ds(base, cpt)], rows_v)
        pltpu.async_copy(rows_v, xs_hbm.at[i0_v], sem).wait()
        pltpu.async_copy(rows_v, xs_hbm.at[i1_v], sem).wait()

    return k(x, p0, p1)


def _sc_combine(ys, p0, p1):
    n = p0.shape[0]
    d = ys.shape[1]
    cpt = n // 32
    half = cpt // 2
    mesh = plsc.VectorSubcoreMesh(core_axis_name="c", subcore_axis_name="s")

    @functools.partial(
        pl.kernel, mesh=mesh,
        out_type=(jax.ShapeDtypeStruct((n, d), jnp.float32),
                  jax.ShapeDtypeStruct((n, d), jnp.float32)),
        scratch_types=[
            pltpu.VMEM((half,), jnp.int32),
            pltpu.VMEM((half, d), jnp.float32),
            pltpu.SemaphoreType.DMA,
        ],
    )
    def k(ys_hbm, p0_hbm, p1_hbm, g0_hbm, g1_hbm, i_v, buf_v, sem):
        wid = lax.axis_index("s") * 2 + lax.axis_index("c")
        base = wid * cpt
        for c in range(2):
            b2 = base + c * half
            pltpu.sync_copy(p0_hbm.at[pl.ds(b2, half)], i_v)
            pltpu.async_copy(ys_hbm.at[i_v], buf_v, sem).wait()
            pltpu.sync_copy(buf_v, g0_hbm.at[pl.ds(b2, half)])
            pltpu.sync_copy(p1_hbm.at[pl.ds(b2, half)], i_v)
            pltpu.async_copy(ys_hbm.at[i_v], buf_v, sem).wait()
            pltpu.sync_copy(buf_v, g1_hbm.at[pl.ds(b2, half)])

    return k(ys, p0, p1)


def _ragged_hidden(h_in, W, b, g, beta, act, e, nbf, first):
    K, N = W.shape
    nk = K // 1024
    nbe = jnp.maximum(nbf[e], 1)

    def xmap(j, k, nb):
        jj = jnp.where(j < nb[e], j, _NEB)
        return (9 * e + jj, k) if first else (jj, k)

    def omap(j, k, nb):
        return (jnp.where(j < nb[e], j, _NEB), 0)

    def body(nb_ref, x_ref, w_ref, b_ref, g_ref, bt_ref, o_ref, acc):
        k = pl.program_id(1)
        xv = x_ref[...]
        if first:
            xv = xv.astype(jnp.bfloat16)
        prod = jnp.dot(xv, w_ref[...].astype(jnp.bfloat16),
                       preferred_element_type=jnp.float32)

        @pl.when(k == 0)
        def _():
            acc[...] = prod

        @pl.when(k > 0)
        def _():
            acc[...] += prod

        @pl.when(k == nk - 1)
        def _():
            hh = acc[...] + b_ref[...]
            hh = _apply_act(act, hh)
            hh = _layernorm(hh, g_ref[...], bt_ref[...])
            o_ref[...] = hh.astype(jnp.bfloat16)

    grid_spec = pltpu.PrefetchScalarGridSpec(
        num_scalar_prefetch=1,
        grid=(nbe, nk),
        in_specs=[
            pl.BlockSpec((_B, 1024), xmap),
            pl.BlockSpec((1024, N), lambda j, k, nb: (k, 0)),
            pl.BlockSpec((1, N), lambda j, k, nb: (0, 0)),
            pl.BlockSpec((1, N), lambda j, k, nb: (0, 0)),
            pl.BlockSpec((1, N), lambda j, k, nb: (0, 0)),
        ],
        out_specs=pl.BlockSpec((_B, N), omap),
        scratch_shapes=[pltpu.VMEM((_B, N), jnp.float32)],
    )
    return pl.pallas_call(
        body,
        grid_spec=grid_spec,
        out_shape=jax.ShapeDtypeStruct((_ECAP, N), jnp.bfloat16),
    )(nbf, h_in, W, b.reshape(1, N), g.reshape(1, N), beta.reshape(1, N))


def _ragged_hidden_final(ys_prev, h_in, l1, l2, act, e, nbf, first):
    W1, W2 = l1['W'], l2['W']
    K, N = W1.shape
    D = W2.shape[1]
    nk = K // 1024
    nbe = jnp.maximum(nbf[e], 1)

    def xmap(j, k, nb):
        jj = jnp.where(j < nb[e], j, _NEB)
        return (9 * e + jj, k) if first else (jj, k)

    def omap(j, k, nb):
        return (9 * e + jnp.where(j < nb[e], j, _NEB), 0)

    def body(nb_ref, *refs):
        if ys_prev is None:
            (x_ref, w1_ref, b1_ref, g1_ref, t1_ref,
             w2_ref, b2_ref, g2_ref, t2_ref, o_ref, acc) = refs
        else:
            (_yp, x_ref, w1_ref, b1_ref, g1_ref, t1_ref,
             w2_ref, b2_ref, g2_ref, t2_ref, o_ref, acc) = refs
        k = pl.program_id(1)
        xv = x_ref[...]
        if first:
            xv = xv.astype(jnp.bfloat16)
        prod = jnp.dot(xv, w1_ref[...].astype(jnp.bfloat16),
                       preferred_element_type=jnp.float32)

        @pl.when(k == 0)
        def _():
            acc[...] = prod

        @pl.when(k > 0)
        def _():
            acc[...] += prod

        @pl.when(k == nk - 1)
        def _():
            hh = acc[...] + b1_ref[...]
            hh = _apply_act(act, hh)
            hh = _layernorm(hh, g1_ref[...], t1_ref[...])
            h2 = jnp.dot(hh.astype(jnp.bfloat16),
                         w2_ref[...].astype(jnp.bfloat16),
                         preferred_element_type=jnp.float32) + b2_ref[...]
            o_ref[...] = _layernorm(h2, g2_ref[...], t2_ref[...])

    in_specs = [
        pl.BlockSpec((_B, 1024), xmap),
        pl.BlockSpec((1024, N), lambda j, k, nb: (k, 0)),
        pl.BlockSpec((1, N), lambda j, k, nb: (0, 0)),
        pl.BlockSpec((1, N), lambda j, k, nb: (0, 0)),
        pl.BlockSpec((1, N), lambda j, k, nb: (0, 0)),
        pl.BlockSpec((N, D), lambda j, k, nb: (0, 0)),
        pl.BlockSpec((1, D), lambda j, k, nb: (0, 0)),
        pl.BlockSpec((1, D), lambda j, k, nb: (0, 0)),
        pl.BlockSpec((1, D), lambda j, k, nb: (0, 0)),
    ]
    args = [nbf, h_in, W1, l1['b'].reshape(1, N), l1['g'].reshape(1, N),
            l1['beta'].reshape(1, N), W2, l2['b'].reshape(1, D),
            l2['g'].reshape(1, D), l2['beta'].reshape(1, D)]
    aliases = {}
    if ys_prev is not None:
        in_specs.insert(0, pl.BlockSpec(memory_space=pl.ANY))
        args.insert(1, ys_prev)
        aliases = {1: 0}
    grid_spec = pltpu.PrefetchScalarGridSpec(
        num_scalar_prefetch=1,
        grid=(nbe, nk),
        in_specs=in_specs,
        out_specs=pl.BlockSpec((_B, D), omap),
        scratch_shapes=[pltpu.VMEM((_B, N), jnp.float32)],
    )
    return pl.pallas_call(
        body,
        grid_spec=grid_spec,
        out_shape=jax.ShapeDtypeStruct((_SLOTS, D), jnp.float32),
        input_output_aliases=aliases,
    )(*args)


def _mix_body(v_ref, a_ref, b_ref, o_ref):
    v = v_ref[...]
    o_ref[...] = v[:, 0:1] * a_ref[...] + v[:, 1:2] * b_ref[...]


def _mix(val2, g0, g1):
    n, d = g0.shape
    bt = 512
    return pl.pallas_call(
        _mix_body,
        grid=(n // bt,),
        in_specs=[
            pl.BlockSpec((bt, 2), lambda t: (t, 0)),
            pl.BlockSpec((bt, d), lambda t: (t, 0)),
            pl.BlockSpec((bt, d), lambda t: (t, 0)),
        ],
        out_specs=pl.BlockSpec((bt, d), lambda t: (t, 0)),
        out_shape=jax.ShapeDtypeStruct((n, d), jnp.float32),
    )(val2, g0, g1)


def kernel(x, params):
    n = x.shape[0]
    val2, pos, nb8 = _gate_meta(x, params['gate_W'], params['gate_b'])
    p0 = pos[:n].reshape(n)
    p1 = pos[n:].reshape(n)
    nbf = nb8.reshape(_NE)
    xs = _sc_dispatch(x, p0, p1)
    ys = None
    for e in range(_NE):
        act, depth, _hid = _cfg(e)
        L = params['experts'][e]
        if depth == 1:
            ys = _ragged_hidden_final(ys, xs, L[0], L[1], act, e, nbf,
                                      first=True)
        elif depth == 2:
            h = _ragged_hidden(xs, L[0]['W'], L[0]['b'], L[0]['g'],
                               L[0]['beta'], act, e, nbf, first=True)
            ys = _ragged_hidden_final(ys, h, L[1], L[2], act, e, nbf,
                                      first=False)
        else:
            h = _ragged_hidden(xs, L[0]['W'], L[0]['b'], L[0]['g'],
                               L[0]['beta'], act, e, nbf, first=True)
            h = _ragged_hidden(h, L[1]['W'], L[1]['b'], L[1]['g'],
                               L[1]['beta'], act, e, nbf, first=False)
            ys = _ragged_hidden_final(ys, h, L[2], L[3], act, e, nbf,
                                      first=False)
    g0, g1 = _sc_combine(ys, p0, p1)
    return _mix(val2, g0, g1)
